# baseline (speedup 1.0000x reference)
import functools

import jax
import jax.numpy as jnp
from jax import lax
from jax.experimental import pallas as pl
from jax.experimental.pallas import tpu as pltpu
from jax.experimental.pallas import tpu_sc as plsc

_K = 32
_T = 16
_NNZ = 64
_NC = 2
_NS = 16
_NW = _NC * _NS
_SEC = 2048
_ICH = 4


def _sc_gather(user_t4, item_t4, uq, iq):
    bu = uq.shape[0] // _NW
    bi = iq.shape[0] // _NW
    ch = bi // _ICH
    mesh = plsc.VectorSubcoreMesh(core_axis_name="c", subcore_axis_name="s")

    @functools.partial(
        pl.kernel,
        mesh=mesh,
        out_type=[
            jax.ShapeDtypeStruct((uq.shape[0], 128), jnp.float32),
            jax.ShapeDtypeStruct((iq.shape[0], 128), jnp.float32),
        ],
        scratch_types=[
            pltpu.VMEM((bu,), jnp.int32),
            pltpu.VMEM((bu, 128), jnp.float32),
            pltpu.VMEM((ch,), jnp.int32),
            pltpu.VMEM((ch, 128), jnp.float32),
            pltpu.SemaphoreType.DMA,
        ],
    )
    def gather_kernel(ut_hbm, it_hbm, uq_hbm, iq_hbm, uout_hbm, iout_hbm,
                      uidx_v, urows_v, iidx_v, irows_v, sem):
        wid = lax.axis_index("s") * _NC + lax.axis_index("c")
        ub = wid * bu
        pltpu.sync_copy(uq_hbm.at[pl.ds(ub, bu)], uidx_v)
        pltpu.async_copy(ut_hbm.at[uidx_v], urows_v, sem).wait()
        pltpu.sync_copy(urows_v, uout_hbm.at[pl.ds(ub, bu)])
        for j in range(_ICH):
            ib = wid * bi + j * ch
            pltpu.sync_copy(iq_hbm.at[pl.ds(ib, ch)], iidx_v)
            pltpu.async_copy(it_hbm.at[iidx_v], irows_v, sem).wait()
            pltpu.sync_copy(irows_v, iout_hbm.at[pl.ds(ib, ch)])

    return gather_kernel(user_t4, item_t4, uq, iq)


def _cbuild_body(wraw_ref, woff_ref, rows_ref, vals_ref, c_ref):
    ow = woff_ref[...]
    w = jnp.zeros((_NNZ, _K), jnp.float32)
    for c in range(4):
        w = w + jnp.where(ow == c, wraw_ref[:, _K * c:_K * (c + 1)],
                          jnp.float32(0.0))
    tag = lax.broadcasted_iota(jnp.int32, (_T, _NNZ), 0)
    S = jnp.where(tag == rows_ref[...], vals_ref[...], jnp.float32(0.0))
    c_ref[...] = lax.dot_general(
        S, w, (((1,), (0,)), ((), ())),
        preferred_element_type=jnp.float32,
        precision=lax.Precision.HIGHEST)


def _main_body(u_ref, vp_ref, vn_ref, offu_ref, offp_ref, offn_ref, c_ref,
               z_ref):
    C = c_ref[...]
    Crep = jnp.concatenate([C, C, C, C], axis=1)
    Crep = jnp.concatenate([Crep, Crep, Crep, Crep], axis=0)
    kk = lax.broadcasted_iota(jnp.int32, (4 * _T, 128), 0) // _T
    ll = lax.broadcasted_iota(jnp.int32, (4 * _T, 128), 1) // _K
    C4 = jnp.where(kk == ll, Crep, jnp.float32(0.0))

    dims = (((1,), (1,)), ((), ()))
    mm = functools.partial(lax.dot_general, dimension_numbers=dims,
                           preferred_element_type=jnp.float32,
                           precision=lax.Precision.HIGHEST)
    A4 = mm(u_ref[...], C4)
    P4 = mm(vp_ref[...], C4)
    N4 = mm(vn_ref[...], C4)

    def fold(x4, off, t):
        acc = jnp.zeros((128, _T), jnp.float32)
        for c in range(4):
            acc = acc + jnp.where(off == c,
                                  x4[128 * t:128 * (t + 1),
                                     _T * c:_T * (c + 1)],
                                  jnp.float32(0.0))
        return acc

    for t in range(_SEC // 128):
        ou = offu_ref[0, :, t:t + 1]
        op = offp_ref[0, :, t:t + 1]
        on = offn_ref[0, :, t:t + 1]
        a = fold(A4, ou, t)
        b = fold(P4, op, t) - fold(N4, on, t)
        z_ref[128 * t:128 * (t + 1), :] = jnp.sum(a * b, axis=1,
                                                  keepdims=True)


def kernel(samples, neg_item, user_table, item_table, rows, cols, vals):
    B = samples.shape[0]
    nu = user_table.shape[0]
    ni = item_table.shape[0]
    user_t4 = user_table.reshape(nu // 4, 128)
    item_t4 = item_table.reshape(ni // 4, 128)

    user_idx = samples[:, 0]
    item_idx = jnp.concatenate([
        cols, jnp.zeros((_SEC - _NNZ,), dtype=cols.dtype),
        samples[:, 1], neg_item,
    ])
    NI = item_idx.shape[0]

    uq = user_idx >> 2
    iq = item_idx >> 2
    spb = _SEC // 128
    offTu = (user_idx & 3).reshape(B // _SEC, spb, 128).transpose(0, 2, 1)
    offTi = (item_idx & 3).reshape(NI // _SEC, spb, 128).transpose(0, 2, 1)
    woff = (cols & 3).reshape(_NNZ, 1)

    raw_u, raw_i = _sc_gather(user_t4, item_t4, uq, iq)

    C = pl.pallas_call(
        _cbuild_body,
        grid=(1,),
        out_shape=jax.ShapeDtypeStruct((_T, _K), jnp.float32),
        in_specs=[
            pl.BlockSpec((_NNZ, 128), lambda g: (0, 0)),
            pl.BlockSpec((_NNZ, 1), lambda g: (0, 0)),
            pl.BlockSpec((1, _NNZ), lambda g: (0, 0)),
            pl.BlockSpec((1, _NNZ), lambda g: (0, 0)),
        ],
        out_specs=pl.BlockSpec((_T, _K), lambda g: (0, 0)),
    )(raw_i, woff, rows.reshape(1, _NNZ), vals.reshape(1, _NNZ))

    nsec = B // _SEC
    z = pl.pallas_call(
        _main_body,
        grid=(nsec,),
        out_shape=jax.ShapeDtypeStruct((B, 1), jnp.float32),
        in_specs=[
            pl.BlockSpec((_SEC, 128), lambda g: (g, 0)),
            pl.BlockSpec((_SEC, 128), lambda g: (g + 1, 0)),
            pl.BlockSpec((_SEC, 128), lambda g: (g + 1 + nsec, 0)),
            pl.BlockSpec((1, 128, spb), lambda g: (g, 0, 0)),
            pl.BlockSpec((1, 128, spb), lambda g: (g + 1, 0, 0)),
            pl.BlockSpec((1, 128, spb), lambda g: (g + 1 + nsec, 0, 0)),
            pl.BlockSpec((_T, _K), lambda g: (0, 0)),
        ],
        out_specs=pl.BlockSpec((_SEC, 1), lambda g: (g, 0)),
    )(raw_u, raw_i, raw_i, offTu, offTi, offTi, C)
    return z


# speedup vs baseline: 1.5204x; 1.5204x over previous
import functools

import jax
import jax.numpy as jnp
from jax import lax
from jax.experimental import pallas as pl
from jax.experimental.pallas import tpu as pltpu
from jax.experimental.pallas import tpu_sc as plsc

_K = 32
_T = 16
_NNZ = 64
_NC = 2
_NS = 16
_NW = _NC * _NS
_SEC = 2048
_UCH = 2
_ICH = 4


def _sc_gather(user_table, item_table, uidx, iidx):
    bu = uidx.shape[0] // _NW
    bi = iidx.shape[0] // _NW
    chu = bu // _UCH
    chi = bi // _ICH
    mesh = plsc.VectorSubcoreMesh(core_axis_name="c", subcore_axis_name="s")

    @functools.partial(
        pl.kernel,
        mesh=mesh,
        out_type=[
            jax.ShapeDtypeStruct((uidx.shape[0], _K), jnp.float32),
            jax.ShapeDtypeStruct((iidx.shape[0], _K), jnp.float32),
        ],
        scratch_types=[
            pltpu.SMEM((max(chu, chi),), jnp.int32),
            pltpu.VMEM((max(chu, chi),), jnp.int32),
            pltpu.VMEM((max(chu, chi), _K), jnp.float32),
            pltpu.SemaphoreType.DMA,
        ],
    )
    def gather_kernel(ut_hbm, it_hbm, uq_hbm, iq_hbm, uout_hbm, iout_hbm,
                      idx_s, idx_v, rows_v, sem):
        wid = lax.axis_index("s") * _NC + lax.axis_index("c")

        def do_chunk(tab, idx_hbm, out_hbm, base, n):
            pltpu.sync_copy(idx_hbm.at[pl.ds(base, n)], idx_v.at[pl.ds(0, n)])

            @pl.loop(0, n, step=16)
            def _(r):
                vec = idx_v[pl.ds(r, 16)]
                for l in range(16):
                    pltpu.async_copy(tab.at[pl.ds(vec[l], 1)],
                                     rows_v.at[pl.ds(r + l, 1)], sem)

            pltpu.make_async_copy(tab.at[pl.ds(0, n)],
                                  rows_v.at[pl.ds(0, n)], sem).wait()
            pltpu.sync_copy(rows_v.at[pl.ds(0, n)],
                            out_hbm.at[pl.ds(base, n)])

        for j in range(_UCH):
            do_chunk(ut_hbm, uq_hbm, uout_hbm, wid * bu + j * chu, chu)
        for j in range(_ICH):
            do_chunk(it_hbm, iq_hbm, iout_hbm, wid * bi + j * chi, chi)

    return gather_kernel(user_table, item_table, uidx, iidx)


def _cbuild_body(wraw_ref, rows_ref, vals_ref, c_ref):
    tag = lax.broadcasted_iota(jnp.int32, (_T, _NNZ), 0)
    S = jnp.where(tag == rows_ref[...], vals_ref[...], jnp.float32(0.0))
    c_ref[...] = lax.dot_general(
        S, wraw_ref[...], (((1,), (0,)), ((), ())),
        preferred_element_type=jnp.float32,
        precision=lax.Precision.HIGHEST)


def _main_body(u_ref, vp_ref, vn_ref, c_ref, z_ref):
    C = c_ref[...]
    dims = (((1,), (1,)), ((), ()))
    mm = functools.partial(lax.dot_general, dimension_numbers=dims,
                           preferred_element_type=jnp.float32,
                           precision=lax.Precision.HIGHEST)
    a = mm(u_ref[...], C)
    b = mm(vp_ref[...] - vn_ref[...], C)
    z_ref[...] = jnp.sum(a * b, axis=1, keepdims=True)


def kernel(samples, neg_item, user_table, item_table, rows, cols, vals):
    B = samples.shape[0]
    user_idx = samples[:, 0]
    item_idx = jnp.concatenate([
        cols, jnp.zeros((_SEC - _NNZ,), dtype=cols.dtype),
        samples[:, 1], neg_item,
    ])
    NI = item_idx.shape[0]

    raw_u, raw_i = _sc_gather(user_table, item_table, user_idx, item_idx)

    C = pl.pallas_call(
        _cbuild_body,
        grid=(1,),
        out_shape=jax.ShapeDtypeStruct((_T, _K), jnp.float32),
        in_specs=[
            pl.BlockSpec((_NNZ, _K), lambda g: (0, 0)),
            pl.BlockSpec((1, _NNZ), lambda g: (0, 0)),
            pl.BlockSpec((1, _NNZ), lambda g: (0, 0)),
        ],
        out_specs=pl.BlockSpec((_T, _K), lambda g: (0, 0)),
    )(raw_i, rows.reshape(1, _NNZ), vals.reshape(1, _NNZ))

    nsec = B // _SEC
    z = pl.pallas_call(
        _main_body,
        grid=(nsec,),
        out_shape=jax.ShapeDtypeStruct((B, 1), jnp.float32),
        in_specs=[
            pl.BlockSpec((_SEC, _K), lambda g: (g, 0)),
            pl.BlockSpec((_SEC, _K), lambda g: (g + 1, 0)),
            pl.BlockSpec((_SEC, _K), lambda g: (g + 1 + nsec, 0)),
            pl.BlockSpec((_T, _K), lambda g: (0, 0)),
        ],
        out_specs=pl.BlockSpec((_SEC, 1), lambda g: (g, 0)),
    )(raw_u, raw_i, raw_i, C)
    return z
